# baseline (speedup 1.0000x reference)
import functools

import jax
import jax.numpy as jnp
from jax import lax
from jax.experimental import pallas as pl
from jax.experimental.pallas import tpu as pltpu
from jax.experimental.pallas import tpu_sc as plsc

N_ENTS = 10000
NREL = 24
H = 128
ZD = 64
EG = 64000
EKG = 320000
NG = 2000
ND = 512
DIS = 512
B = 512
TAU = 0.5

NC = 2
NS = 16
NW = NC * NS
EB = 50
NBK = EKG // (NW * EB)
NBG = EG // (NW * EB)

_SC_MESH = plsc.VectorSubcoreMesh(core_axis_name="c", subcore_axis_name="s")


def _rows_copy(src, dst, s, n):
    per = (n // NS) // 8 * 8
    rem = n - per * NS
    pltpu.sync_copy(src.at[pl.ds(s * per, per)], dst.at[pl.ds(s * per, per)])
    if rem:
        @pl.when(s == NS - 1)
        def _():
            pltpu.sync_copy(src.at[pl.ds(per * NS, rem)], dst.at[pl.ds(per * NS, rem)])


def _rgcn_sc_body(kg_src4, kg_dst4, kg_et4, gs_dst4, ent, rel,
                  zero_agg,
                  agg_out, dkg_out, dgg_out, dgs_out,
                  sidxA, sidxB, didxA, didxB, etvA, etvB,
                  rows0, rows1, rrows0, rrows1, ones128,
                  acc_sh,
                  is0, is1, gs0, gs1, rs0, rs1, ss0, ss1):
    c = lax.axis_index("c")
    s = lax.axis_index("s")
    w = c * NS + s

    _rows_copy(zero_agg, acc_sh, s, N_ENTS)

    def ones_body(i, _):
        r = i // 8
        q = i % 8
        ones128[r, pl.ds(q * 16, 16)] = jnp.full((16,), 1.0, jnp.float32)
        return 0
    lax.fori_loop(0, EB * 8, ones_body, 0)

    plsc.subcore_barrier()

    sidxs = (sidxA, sidxB)
    didxs = (didxA, didxB)
    etvs = (etvA, etvB)
    isems = (is0, is1)
    rows = (rows0, rows1)
    rrows = (rrows0, rrows1)
    gs = (gs0, gs1)
    rs = (rs0, rs1)

    def afetch(b, p):
        pltpu.async_copy(kg_src4.at[w, b, 0], sidxs[p], isems[p])
        pltpu.async_copy(kg_dst4.at[w, b, 0], didxs[p], isems[p])
        pltpu.async_copy(kg_et4.at[w, b, 0], etvs[p], isems[p])

    def wait_idx(p):
        pltpu.make_async_copy(kg_src4.at[0, 0, 0], sidxs[p], isems[p]).wait()
        pltpu.make_async_copy(kg_src4.at[0, 0, 0], didxs[p], isems[p]).wait()
        pltpu.make_async_copy(kg_src4.at[0, 0, 0], etvs[p], isems[p]).wait()

    def issue_gather(p):
        pltpu.async_copy(ent.at[sidxs[p]], rows[p], gs[p])
        pltpu.async_copy(rel.at[etvs[p]], rrows[p], rs[p])

    def wait_gather(p):
        pltpu.make_async_copy(ent.at[sidxs[p]], rows[p], gs[p]).wait()
        pltpu.make_async_copy(rel.at[etvs[p]], rrows[p], rs[p]).wait()

    def multiply(p):
        rp, rrp = rows[p], rrows[p]

        def edge_body(e, _):
            for j in range(H // 16):
                sl = pl.ds(16 * j, 16)
                rp[e, sl] = rp[e, sl] * rrp[e, sl]
            return 0
        lax.fori_loop(0, EB, edge_body, 0)

    ssems = (ss0, ss1)

    def wait_scat(p):
        pltpu.make_async_copy(rows[p], acc_sh.at[didxs[p]], ssems[p]).wait()

    def step(b, p, q):
        wait_gather(p)

        @pl.when(b + 2 < NBK)
        def _():
            afetch(b + 2, p)

        @pl.when(b + 1 < NBK)
        def _():
            wait_idx(q)

            @pl.when(b >= 1)
            def _():
                wait_scat(q)
            issue_gather(q)
        multiply(p)
        pltpu.async_copy(rows[p], acc_sh.at[didxs[p]], ssems[p], add=True)

    afetch(0, 0)
    afetch(1, 1)
    wait_idx(0)
    issue_gather(0)

    def pair_body(j, _):
        step(2 * j, 0, 1)
        step(2 * j + 1, 1, 0)
        return 0
    lax.fori_loop(0, NBK // 2, pair_body, 0)
    wait_scat(0)
    wait_scat(1)

    plsc.subcore_barrier()

    _rows_copy(acc_sh, agg_out.at[c], s, N_ENTS)
    _rows_copy(zero_agg, acc_sh, s, N_ENTS)
    plsc.subcore_barrier()

    def kfetch(b, p):
        pltpu.async_copy(kg_dst4.at[w, b, 0], didxs[p], isems[p])

    def wait_didx(p):
        pltpu.make_async_copy(kg_src4.at[0, 0, 0], didxs[p], isems[p]).wait()

    kfetch(0, 0)
    kfetch(1, 1)

    def kdeg_step(b, p):
        wait_didx(p)
        pltpu.sync_copy(ones128, acc_sh.at[didxs[p]], add=True)

        @pl.when(b + 2 < NBK)
        def _():
            kfetch(b + 2, p)

    def kdeg_pair(j, _):
        kdeg_step(2 * j, 0)
        kdeg_step(2 * j + 1, 1)
        return 0
    lax.fori_loop(0, NBK // 2, kdeg_pair, 0)

    plsc.subcore_barrier()
    _rows_copy(acc_sh, dkg_out.at[c], s, N_ENTS)
    _rows_copy(zero_agg, acc_sh, s, 2 * NG)
    plsc.subcore_barrier()

    NBC = 2 * NBG

    def cfetch(b, p):
        pltpu.async_copy(gs_dst4.at[w, b, 0], didxs[p], isems[p])

    cfetch(0, 0)
    cfetch(1, 1)

    def cdeg_step(b, p):
        wait_didx(p)
        pltpu.sync_copy(ones128, acc_sh.at[didxs[p]], add=True)

        @pl.when(b + 2 < NBC)
        def _():
            cfetch(b + 2, p)

    def cdeg_pair(j, _):
        cdeg_step(2 * j, 0)
        cdeg_step(2 * j + 1, 1)
        return 0
    lax.fori_loop(0, NBC // 2, cdeg_pair, 0)

    plsc.subcore_barrier()
    _rows_copy(acc_sh, dgg_out.at[c], s, NG)

    per = (NG // NS) // 8 * 8
    rem = NG - per * NS
    pltpu.sync_copy(acc_sh.at[pl.ds(NG + s * per, per)],
                    dgs_out.at[c].at[pl.ds(s * per, per)])

    @pl.when(s == NS - 1)
    def _():
        pltpu.sync_copy(acc_sh.at[pl.ds(NG + per * NS, rem)],
                        dgs_out.at[c].at[pl.ds(per * NS, rem)])


def _rgcn_sc(kg_src, kg_dst, kg_et, g_dst, s_dst, ent, rel):
    zero_agg = jnp.zeros((N_ENTS, H), jnp.float32)
    gs_dst = jnp.concatenate([g_dst, s_dst + NG])
    f = pl.kernel(
        _rgcn_sc_body,
        out_type=(
            jax.ShapeDtypeStruct((NC, N_ENTS, H), jnp.float32),
            jax.ShapeDtypeStruct((NC, N_ENTS, H), jnp.float32),
            jax.ShapeDtypeStruct((NC, NG, H), jnp.float32),
            jax.ShapeDtypeStruct((NC, NG, H), jnp.float32),
        ),
        mesh=_SC_MESH,
        scratch_types=[
            pltpu.VMEM((EB,), jnp.int32),
            pltpu.VMEM((EB,), jnp.int32),
            pltpu.VMEM((EB,), jnp.int32),
            pltpu.VMEM((EB,), jnp.int32),
            pltpu.VMEM((EB,), jnp.int32),
            pltpu.VMEM((EB,), jnp.int32),
            pltpu.VMEM((EB, H), jnp.float32),
            pltpu.VMEM((EB, H), jnp.float32),
            pltpu.VMEM((EB, H), jnp.float32),
            pltpu.VMEM((EB, H), jnp.float32),
            pltpu.VMEM((EB, H), jnp.float32),
            pltpu.VMEM_SHARED((N_ENTS, H), jnp.float32),
            pltpu.SemaphoreType.DMA,
            pltpu.SemaphoreType.DMA,
            pltpu.SemaphoreType.DMA,
            pltpu.SemaphoreType.DMA,
            pltpu.SemaphoreType.DMA,
            pltpu.SemaphoreType.DMA,
            pltpu.SemaphoreType.DMA,
            pltpu.SemaphoreType.DMA,
        ],
    )
    return f(kg_src.reshape(NW, NBK, 1, EB), kg_dst.reshape(NW, NBK, 1, EB),
             kg_et.reshape(NW, NBK, 1, EB), gs_dst.reshape(NW, 2 * NBG, 1, EB),
             ent, rel, zero_agg)


def _gcn_agg_body(zg, zs, g_src4, g_dst4, s_src4, s_dst4, zero_ng,
                  u_out,
                  sidxA, sidxB, didxA, didxB, rows0, rows1,
                  accg_sh, accs_sh, is0, is1, gs0, gs1, ss0, ss1):
    c = lax.axis_index("c")
    s = lax.axis_index("s")
    w = c * NS + s

    _rows_copy(zero_ng, accg_sh, s, NG)
    _rows_copy(zero_ng, accs_sh, s, NG)
    plsc.subcore_barrier()

    sidxs = (sidxA, sidxB)
    didxs = (didxA, didxB)
    isems = (is0, is1)
    rows = (rows0, rows1)
    gs = (gs0, gs1)
    ssems = (ss0, ss1)

    def run_graph(src4, dst4, tab, acc):
        def afetch(b, p):
            pltpu.async_copy(src4.at[w, b, 0], sidxs[p], isems[p])
            pltpu.async_copy(dst4.at[w, b, 0], didxs[p], isems[p])

        def wait_idx(p):
            pltpu.make_async_copy(src4.at[0, 0, 0], sidxs[p], isems[p]).wait()
            pltpu.make_async_copy(src4.at[0, 0, 0], didxs[p], isems[p]).wait()

        def issue_gather(p):
            pltpu.async_copy(tab.at[sidxs[p]], rows[p], gs[p])

        def wait_gather(p):
            pltpu.make_async_copy(tab.at[sidxs[p]], rows[p], gs[p]).wait()

        def wait_scat(p):
            pltpu.make_async_copy(rows[p], acc.at[didxs[p]], ssems[p]).wait()

        def step(b, p, q):
            wait_gather(p)

            @pl.when(b + 2 < NBG)
            def _():
                afetch(b + 2, p)

            @pl.when(b + 1 < NBG)
            def _():
                wait_idx(q)

                @pl.when(b >= 1)
                def _():
                    wait_scat(q)
                issue_gather(q)
            pltpu.async_copy(rows[p], acc.at[didxs[p]], ssems[p], add=True)

        afetch(0, 0)
        afetch(1, 1)
        wait_idx(0)
        issue_gather(0)

        def pair_body(j, _):
            step(2 * j, 0, 1)
            step(2 * j + 1, 1, 0)
            return 0
        lax.fori_loop(0, NBG // 2, pair_body, 0)
        wait_scat(0)
        wait_scat(1)

    run_graph(g_src4, g_dst4, zg, accg_sh)
    run_graph(s_src4, s_dst4, zs, accs_sh)

    plsc.subcore_barrier()

    _rows_copy(accg_sh, u_out.at[c, 0], s, NG)
    _rows_copy(accs_sh, u_out.at[c, 1], s, NG)


def _gcn_agg(zg, zs, g_src, g_dst, s_src, s_dst):
    zero_ng = jnp.zeros((NG, H), jnp.float32)
    f = pl.kernel(
        _gcn_agg_body,
        out_type=jax.ShapeDtypeStruct((NC, 2, NG, H), jnp.float32),
        mesh=_SC_MESH,
        scratch_types=[
            pltpu.VMEM((EB,), jnp.int32),
            pltpu.VMEM((EB,), jnp.int32),
            pltpu.VMEM((EB,), jnp.int32),
            pltpu.VMEM((EB,), jnp.int32),
            pltpu.VMEM((EB, H), jnp.float32),
            pltpu.VMEM((EB, H), jnp.float32),
            pltpu.VMEM_SHARED((NG, H), jnp.float32),
            pltpu.VMEM_SHARED((NG, H), jnp.float32),
            pltpu.SemaphoreType.DMA,
            pltpu.SemaphoreType.DMA,
            pltpu.SemaphoreType.DMA,
            pltpu.SemaphoreType.DMA,
            pltpu.SemaphoreType.DMA,
            pltpu.SemaphoreType.DMA,
        ],
    )
    return f(zg, zs, g_src.reshape(NW, NBG, 1, EB), g_dst.reshape(NW, NBG, 1, EB),
             s_src.reshape(NW, NBG, 1, EB), s_dst.reshape(NW, NBG, 1, EB), zero_ng)


def _rgcn_linear_body(agg_ref, deg_ref, ent_ref, wkg_ref, wself_ref, out_ref):
    deg = jnp.maximum(deg_ref[0, :, 0:1] + deg_ref[1, :, 0:1], 1.0)
    agg = (agg_ref[0] + agg_ref[1]) / deg
    h = jnp.dot(agg, wkg_ref[...], preferred_element_type=jnp.float32)
    h = h + jnp.dot(ent_ref[...], wself_ref[...], preferred_element_type=jnp.float32)
    out_ref[...] = jnp.maximum(h, 0.0)


def _rgcn_linear(agg_part, dkg_part, ent_emb, W_kg, W_self):
    grid = 10
    bm = N_ENTS // grid
    return pl.pallas_call(
        _rgcn_linear_body,
        grid=(grid,),
        in_specs=[
            pl.BlockSpec((NC, bm, H), lambda i: (0, i, 0)),
            pl.BlockSpec((NC, bm, H), lambda i: (0, i, 0)),
            pl.BlockSpec((bm, H), lambda i: (i, 0)),
            pl.BlockSpec((H, H), lambda i: (0, 0)),
            pl.BlockSpec((H, H), lambda i: (0, 0)),
        ],
        out_specs=pl.BlockSpec((bm, H), lambda i: (i, 0)),
        out_shape=jax.ShapeDtypeStruct((N_ENTS, H), jnp.float32),
    )(agg_part, dkg_part, ent_emb, W_kg, W_self)


def _rsq_body(dg_ref, ds_ref, og_ref, os_ref):
    og_ref[...] = jax.lax.rsqrt(jnp.maximum(dg_ref[0, :, 0:1] + dg_ref[1, :, 0:1], 1.0))
    os_ref[...] = jax.lax.rsqrt(jnp.maximum(ds_ref[0, :, 0:1] + ds_ref[1, :, 0:1], 1.0))


def _rsq_degs(dgg_part, dgs_part):
    return pl.pallas_call(
        _rsq_body,
        out_shape=(
            jax.ShapeDtypeStruct((NG, 1), jnp.float32),
            jax.ShapeDtypeStruct((NG, 1), jnp.float32),
        ),
    )(dgg_part, dgs_part)


def _pool_scale_body(y2x_ref, h_ref, rg_ref, rs_ref, out_ref):
    blk = y2x_ref[...]
    y = jnp.dot(blk, h_ref[...], preferred_element_type=jnp.float32)
    rs = jnp.clip(jnp.sum(blk, axis=1, keepdims=True), 1e-8, None)
    g = y / rs
    out_ref[0] = g * rg_ref[...]
    out_ref[1] = g * rs_ref[...]


def _pool_g2o_scaled(g2o, kg_h, rsq_g, rsq_s):
    grid = 10
    bm = NG // grid
    return pl.pallas_call(
        _pool_scale_body,
        grid=(grid,),
        in_specs=[
            pl.BlockSpec((bm, N_ENTS), lambda i: (i, 0)),
            pl.BlockSpec((N_ENTS, H), lambda i: (0, 0)),
            pl.BlockSpec((bm, 1), lambda i: (i, 0)),
            pl.BlockSpec((bm, 1), lambda i: (i, 0)),
        ],
        out_specs=pl.BlockSpec((2, bm, H), lambda i: (0, i, 0)),
        out_shape=jax.ShapeDtypeStruct((2, NG, H), jnp.float32),
    )(g2o, kg_h, rsq_g, rsq_s)


def _gcn_mid_body(u1_ref, rg_ref, rs_ref, w1_ref, out_ref):
    w1 = w1_ref[...]
    rg = rg_ref[...]
    rs = rs_ref[...]
    ug = u1_ref[0, 0] + u1_ref[1, 0]
    us = u1_ref[0, 1] + u1_ref[1, 1]
    hg = jnp.maximum(jnp.dot(ug * rg, w1, preferred_element_type=jnp.float32), 0.0)
    hs = jnp.maximum(jnp.dot(us * rs, w1, preferred_element_type=jnp.float32), 0.0)
    out_ref[0] = hg * rg
    out_ref[1] = hs * rs


def _gcn_mid(u1_part, rsq_g, rsq_s, W1):
    return pl.pallas_call(
        _gcn_mid_body,
        out_shape=jax.ShapeDtypeStruct((2, NG, H), jnp.float32),
    )(u1_part, rsq_g, rsq_s, W1)


def _leaky(x):
    return jnp.where(x >= 0, x, 0.01 * x)


def _l2n(x):
    n = jnp.sqrt(jnp.sum(x * x, axis=1, keepdims=True))
    return x / jnp.maximum(n, 1e-12)


def _nce_loss(x1, x2):
    x1 = _l2n(x1)
    x2 = _l2n(x2)
    sm = jnp.exp(jnp.dot(x1, x2.T, preferred_element_type=jnp.float32) / TAU)
    s = jnp.sum(sm, axis=1)
    n = x1.shape[0]
    eye = (jax.lax.broadcasted_iota(jnp.int32, (n, n), 0)
           == jax.lax.broadcasted_iota(jnp.int32, (n, n), 1))
    pos = jnp.sum(jnp.where(eye, sm, 0.0), axis=1)
    return -jnp.mean(jnp.log(pos / s + 1e-8))


def _final_body(u2_ref, rg_ref, rs_ref, w2_ref, d2g_ref, dh2_ref, wp1_ref,
                bp1_ref, wp2_ref, bp2_ref, idx_ref, st_ref, lam_ref,
                dz_ref, demb_ref, sim_ref, loss_ref):
    w2 = w2_ref[...]
    ug = u2_ref[0, 0] + u2_ref[1, 0]
    us = u2_ref[0, 1] + u2_ref[1, 1]
    g_h1 = jnp.dot(ug * rg_ref[...], w2, preferred_element_type=jnp.float32)
    g_h2 = jnp.dot(us * rs_ref[...], w2, preferred_element_type=jnp.float32)
    g_havg = 0.5 * (g_h1 + g_h2)

    d2g = d2g_ref[...]
    rs = jnp.clip(jnp.sum(d2g, axis=1, keepdims=True), 1e-8, None)
    d_h1 = jnp.dot(d2g, g_havg, preferred_element_type=jnp.float32) / rs
    d_emb = jnp.dot(d_h1, wp1_ref[...], preferred_element_type=jnp.float32) + bp1_ref[...]
    d_z2p = jnp.dot(dh2_ref[...], wp2_ref[...], preferred_element_type=jnp.float32) + bp2_ref[...]

    idx = idx_ref[...]
    col = jax.lax.broadcasted_iota(jnp.int32, (B, ND), 1)
    oh1 = (idx[:, 0:1] == col).astype(jnp.float32)
    oh2 = (idx[:, 1:2] == col).astype(jnp.float32)
    d_z1 = _leaky(jnp.dot(oh1, d_emb, preferred_element_type=jnp.float32))
    d_z2 = _leaky(jnp.dot(oh2, d_z2p, preferred_element_type=jnp.float32))
    d_z = 0.5 * (d_z1 + d_z2)

    sim = _l2n(jnp.dot(d_z, d_z.T, preferred_element_type=jnp.float32))
    logits = sim + 1e-8
    mx = jnp.max(logits, axis=1, keepdims=True)
    lse = mx + jnp.log(jnp.sum(jnp.exp(logits - mx), axis=1, keepdims=True))
    logp = logits - lse
    colb = jax.lax.broadcasted_iota(jnp.int32, (B, B), 1)
    oht = (st_ref[...][:, 0:1] == colb)
    l_p = -jnp.mean(jnp.sum(jnp.where(oht, logp, 0.0), axis=1))

    l_g = _nce_loss(g_h1, g_h2)
    l_d = _nce_loss(d_z1, d_z2)
    lam = lam_ref[...]
    loss = l_p + lam[0, 0] * l_g + lam[0, 1] * l_d

    dz_ref[...] = d_z
    demb_ref[...] = d_emb
    sim_ref[...] = sim
    loss_ref[...] = jnp.reshape(loss, (1, 1))


def _final_block(u2_part, rsq_g, rsq_s, W2, d2g, d_h2, Wp1, bp1, Wp2, bp2,
                 index, similarity_true, lam):
    return pl.pallas_call(
        _final_body,
        out_shape=(
            jax.ShapeDtypeStruct((B, ZD), jnp.float32),
            jax.ShapeDtypeStruct((ND, ZD), jnp.float32),
            jax.ShapeDtypeStruct((B, B), jnp.float32),
            jax.ShapeDtypeStruct((1, 1), jnp.float32),
        ),
    )(u2_part, rsq_g, rsq_s, W2, d2g, d_h2,
      Wp1, bp1.reshape(1, ZD), Wp2, bp2.reshape(1, ZD),
      index, similarity_true.reshape(B, 1), lam)


def kernel(g_edge_index, g_svd_edge_index, kg_edge_index, kg_edge_type, g2o,
           d2g, d_h2, index, similarity_true, lam_1, lam_2, ent_emb, rel_emb,
           W_kg, W_self, W1, W2, Wp1, bp1, Wp2, bp2):
    g_src, g_dst = g_edge_index[0], g_edge_index[1]
    s_src, s_dst = g_svd_edge_index[0], g_svd_edge_index[1]
    agg_part, dkg_part, dgg_part, dgs_part = _rgcn_sc(
        kg_edge_index[0], kg_edge_index[1], kg_edge_type, g_dst, s_dst,
        ent_emb, rel_emb)

    kg_h = _rgcn_linear(agg_part, dkg_part, ent_emb, W_kg, W_self)
    rsq_g, rsq_s = _rsq_degs(dgg_part, dgs_part)

    z0 = _pool_g2o_scaled(g2o, kg_h, rsq_g, rsq_s)
    u1_part = _gcn_agg(z0[0], z0[1], g_src, g_dst, s_src, s_dst)
    z1 = _gcn_mid(u1_part, rsq_g, rsq_s, W1)
    u2_part = _gcn_agg(z1[0], z1[1], g_src, g_dst, s_src, s_dst)

    lam = jnp.stack([jnp.squeeze(lam_1), jnp.squeeze(lam_2)]).reshape(1, 2)
    d_z, d_emb, sim, loss = _final_block(
        u2_part, rsq_g, rsq_s, W2, d2g, d_h2, Wp1, bp1, Wp2, bp2,
        index, similarity_true, lam)
    return (d_z, d_emb, sim, jnp.reshape(loss, ()))


# speedup vs baseline: 1.0254x; 1.0254x over previous
import functools

import jax
import jax.numpy as jnp
from jax import lax
from jax.experimental import pallas as pl
from jax.experimental.pallas import tpu as pltpu
from jax.experimental.pallas import tpu_sc as plsc

N_ENTS = 10000
NREL = 24
H = 128
ZD = 64
EG = 64000
EKG = 320000
NG = 2000
ND = 512
DIS = 512
B = 512
TAU = 0.5

NC = 2
NS = 16
NW = NC * NS
EB = 50
NBK = EKG // (NW * EB)
NBG = EG // (NW * EB)
EBD = 100
NBKD = EKG // (NW * EBD)
NBGD = EG // (NW * EBD)

_SC_MESH = plsc.VectorSubcoreMesh(core_axis_name="c", subcore_axis_name="s")


def _rows_copy(src, dst, s, n):
    per = (n // NS) // 8 * 8
    rem = n - per * NS
    pltpu.sync_copy(src.at[pl.ds(s * per, per)], dst.at[pl.ds(s * per, per)])
    if rem:
        @pl.when(s == NS - 1)
        def _():
            pltpu.sync_copy(src.at[pl.ds(per * NS, rem)], dst.at[pl.ds(per * NS, rem)])


def _rgcn_sc_body(kg_src4, kg_dst4, kg_et4, kg_dstd4, gs_dstd4, ent, rel,
                  zero_agg,
                  agg_out, dkg_out, dgg_out, dgs_out,
                  sidxA, sidxB, didxA, didxB, etvA, etvB, ddxA, ddxB,
                  rows0, rows1, rrows0, rrows1, ones128,
                  acc_sh,
                  is0, is1, gs0, gs1, rs0, rs1, ss0, ss1):
    c = lax.axis_index("c")
    s = lax.axis_index("s")
    w = c * NS + s

    _rows_copy(zero_agg, acc_sh, s, N_ENTS)

    def ones_body(i, _):
        r = i // 8
        q = i % 8
        ones128[r, pl.ds(q * 16, 16)] = jnp.full((16,), 1.0, jnp.float32)
        return 0
    lax.fori_loop(0, EBD * 8, ones_body, 0)

    plsc.subcore_barrier()

    sidxs = (sidxA, sidxB)
    didxs = (didxA, didxB)
    etvs = (etvA, etvB)
    isems = (is0, is1)
    rows = (rows0, rows1)
    rrows = (rrows0, rrows1)
    gs = (gs0, gs1)
    rs = (rs0, rs1)

    def afetch(b, p):
        pltpu.async_copy(kg_src4.at[w, b, 0], sidxs[p], isems[p])
        pltpu.async_copy(kg_dst4.at[w, b, 0], didxs[p], isems[p])
        pltpu.async_copy(kg_et4.at[w, b, 0], etvs[p], isems[p])

    def wait_idx(p):
        pltpu.make_async_copy(kg_src4.at[0, 0, 0], sidxs[p], isems[p]).wait()
        pltpu.make_async_copy(kg_src4.at[0, 0, 0], didxs[p], isems[p]).wait()
        pltpu.make_async_copy(kg_src4.at[0, 0, 0], etvs[p], isems[p]).wait()

    def issue_gather(p):
        pltpu.async_copy(ent.at[sidxs[p]], rows[p], gs[p])
        pltpu.async_copy(rel.at[etvs[p]], rrows[p], rs[p])

    def wait_gather(p):
        pltpu.make_async_copy(ent.at[sidxs[p]], rows[p], gs[p]).wait()
        pltpu.make_async_copy(rel.at[etvs[p]], rrows[p], rs[p]).wait()

    def multiply(p):
        rp, rrp = rows[p], rrows[p]

        def edge_body(i, _):
            e = 2 * i
            for d in range(2):
                for j in range(H // 16):
                    sl = pl.ds(16 * j, 16)
                    rp[e + d, sl] = rp[e + d, sl] * rrp[e + d, sl]
            return 0
        lax.fori_loop(0, EB // 2, edge_body, 0)

    ssems = (ss0, ss1)

    def wait_scat(p):
        pltpu.make_async_copy(rows[p], acc_sh.at[didxs[p]], ssems[p]).wait()

    def step(b, p, q):
        wait_gather(p)

        @pl.when(b + 2 < NBK)
        def _():
            afetch(b + 2, p)

        @pl.when(b + 1 < NBK)
        def _():
            wait_idx(q)

            @pl.when(b >= 1)
            def _():
                wait_scat(q)
            issue_gather(q)
        multiply(p)
        pltpu.async_copy(rows[p], acc_sh.at[didxs[p]], ssems[p], add=True)

    afetch(0, 0)
    afetch(1, 1)
    wait_idx(0)
    issue_gather(0)

    def pair_body(j, _):
        step(2 * j, 0, 1)
        step(2 * j + 1, 1, 0)
        return 0
    lax.fori_loop(0, NBK // 2, pair_body, 0)
    wait_scat(0)
    wait_scat(1)

    plsc.subcore_barrier()

    _rows_copy(acc_sh, agg_out.at[c], s, N_ENTS)
    _rows_copy(zero_agg, acc_sh, s, N_ENTS)
    plsc.subcore_barrier()

    ddxs = (ddxA, ddxB)

    def kfetch(b, p):
        pltpu.async_copy(kg_dstd4.at[w, b, 0], ddxs[p], isems[p])

    def wait_didx(p):
        pltpu.make_async_copy(kg_dstd4.at[0, 0, 0], ddxs[p], isems[p]).wait()

    kfetch(0, 0)
    kfetch(1, 1)

    def kdeg_step(b, p):
        wait_didx(p)
        pltpu.sync_copy(ones128, acc_sh.at[ddxs[p]], add=True)

        @pl.when(b + 2 < NBKD)
        def _():
            kfetch(b + 2, p)

    def kdeg_pair(j, _):
        kdeg_step(2 * j, 0)
        kdeg_step(2 * j + 1, 1)
        return 0
    lax.fori_loop(0, NBKD // 2, kdeg_pair, 0)

    plsc.subcore_barrier()
    _rows_copy(acc_sh, dkg_out.at[c], s, N_ENTS)
    _rows_copy(zero_agg, acc_sh, s, 2 * NG)
    plsc.subcore_barrier()

    NBC = 2 * NBGD

    def cfetch(b, p):
        pltpu.async_copy(gs_dstd4.at[w, b, 0], ddxs[p], isems[p])

    cfetch(0, 0)
    cfetch(1, 1)

    def cdeg_step(b, p):
        wait_didx(p)
        pltpu.sync_copy(ones128, acc_sh.at[ddxs[p]], add=True)

        @pl.when(b + 2 < NBC)
        def _():
            cfetch(b + 2, p)

    def cdeg_pair(j, _):
        cdeg_step(2 * j, 0)
        cdeg_step(2 * j + 1, 1)
        return 0
    lax.fori_loop(0, NBC // 2, cdeg_pair, 0)

    plsc.subcore_barrier()
    _rows_copy(acc_sh, dgg_out.at[c], s, NG)

    per = (NG // NS) // 8 * 8
    rem = NG - per * NS
    pltpu.sync_copy(acc_sh.at[pl.ds(NG + s * per, per)],
                    dgs_out.at[c].at[pl.ds(s * per, per)])

    @pl.when(s == NS - 1)
    def _():
        pltpu.sync_copy(acc_sh.at[pl.ds(NG + per * NS, rem)],
                        dgs_out.at[c].at[pl.ds(per * NS, rem)])


def _rgcn_sc(kg_src, kg_dst, kg_et, g_dst, s_dst, ent, rel):
    zero_agg = jnp.zeros((N_ENTS, H), jnp.float32)
    gs_dst = jnp.concatenate([g_dst, s_dst + NG])
    f = pl.kernel(
        _rgcn_sc_body,
        out_type=(
            jax.ShapeDtypeStruct((NC, N_ENTS, H), jnp.float32),
            jax.ShapeDtypeStruct((NC, N_ENTS, H), jnp.float32),
            jax.ShapeDtypeStruct((NC, NG, H), jnp.float32),
            jax.ShapeDtypeStruct((NC, NG, H), jnp.float32),
        ),
        mesh=_SC_MESH,
        scratch_types=[
            pltpu.VMEM((EB,), jnp.int32),
            pltpu.VMEM((EB,), jnp.int32),
            pltpu.VMEM((EB,), jnp.int32),
            pltpu.VMEM((EB,), jnp.int32),
            pltpu.VMEM((EB,), jnp.int32),
            pltpu.VMEM((EB,), jnp.int32),
            pltpu.VMEM((EBD,), jnp.int32),
            pltpu.VMEM((EBD,), jnp.int32),
            pltpu.VMEM((EB, H), jnp.float32),
            pltpu.VMEM((EB, H), jnp.float32),
            pltpu.VMEM((EB, H), jnp.float32),
            pltpu.VMEM((EB, H), jnp.float32),
            pltpu.VMEM((EBD, H), jnp.float32),
            pltpu.VMEM_SHARED((N_ENTS, H), jnp.float32),
            pltpu.SemaphoreType.DMA,
            pltpu.SemaphoreType.DMA,
            pltpu.SemaphoreType.DMA,
            pltpu.SemaphoreType.DMA,
            pltpu.SemaphoreType.DMA,
            pltpu.SemaphoreType.DMA,
            pltpu.SemaphoreType.DMA,
            pltpu.SemaphoreType.DMA,
        ],
    )
    return f(kg_src.reshape(NW, NBK, 1, EB), kg_dst.reshape(NW, NBK, 1, EB),
             kg_et.reshape(NW, NBK, 1, EB), kg_dst.reshape(NW, NBKD, 1, EBD),
             gs_dst.reshape(NW, 2 * NBGD, 1, EBD), ent, rel, zero_agg)


def _gcn_agg_body(zg, zs, g_src4, g_dst4, s_src4, s_dst4, zero_ng,
                  u_out,
                  sidxA, sidxB, didxA, didxB, rows0, rows1,
                  accg_sh, accs_sh, is0, is1, gs0, gs1, ss0, ss1):
    c = lax.axis_index("c")
    s = lax.axis_index("s")
    w = c * NS + s

    _rows_copy(zero_ng, accg_sh, s, NG)
    _rows_copy(zero_ng, accs_sh, s, NG)
    plsc.subcore_barrier()

    sidxs = (sidxA, sidxB)
    didxs = (didxA, didxB)
    isems = (is0, is1)
    rows = (rows0, rows1)
    gs = (gs0, gs1)
    ssems = (ss0, ss1)

    def run_graph(src4, dst4, tab, acc):
        def afetch(b, p):
            pltpu.async_copy(src4.at[w, b, 0], sidxs[p], isems[p])
            pltpu.async_copy(dst4.at[w, b, 0], didxs[p], isems[p])

        def wait_idx(p):
            pltpu.make_async_copy(src4.at[0, 0, 0], sidxs[p], isems[p]).wait()
            pltpu.make_async_copy(src4.at[0, 0, 0], didxs[p], isems[p]).wait()

        def issue_gather(p):
            pltpu.async_copy(tab.at[sidxs[p]], rows[p], gs[p])

        def wait_gather(p):
            pltpu.make_async_copy(tab.at[sidxs[p]], rows[p], gs[p]).wait()

        def wait_scat(p):
            pltpu.make_async_copy(rows[p], acc.at[didxs[p]], ssems[p]).wait()

        def step(b, p, q):
            wait_gather(p)

            @pl.when(b + 2 < NBG)
            def _():
                afetch(b + 2, p)

            @pl.when(b + 1 < NBG)
            def _():
                wait_idx(q)

                @pl.when(b >= 1)
                def _():
                    wait_scat(q)
                issue_gather(q)
            pltpu.async_copy(rows[p], acc.at[didxs[p]], ssems[p], add=True)

        afetch(0, 0)
        afetch(1, 1)
        wait_idx(0)
        issue_gather(0)

        def pair_body(j, _):
            step(2 * j, 0, 1)
            step(2 * j + 1, 1, 0)
            return 0
        lax.fori_loop(0, NBG // 2, pair_body, 0)
        wait_scat(0)
        wait_scat(1)

    run_graph(g_src4, g_dst4, zg, accg_sh)
    run_graph(s_src4, s_dst4, zs, accs_sh)

    plsc.subcore_barrier()

    _rows_copy(accg_sh, u_out.at[c, 0], s, NG)
    _rows_copy(accs_sh, u_out.at[c, 1], s, NG)


def _gcn_agg(zg, zs, g_src, g_dst, s_src, s_dst):
    zero_ng = jnp.zeros((NG, H), jnp.float32)
    f = pl.kernel(
        _gcn_agg_body,
        out_type=jax.ShapeDtypeStruct((NC, 2, NG, H), jnp.float32),
        mesh=_SC_MESH,
        scratch_types=[
            pltpu.VMEM((EB,), jnp.int32),
            pltpu.VMEM((EB,), jnp.int32),
            pltpu.VMEM((EB,), jnp.int32),
            pltpu.VMEM((EB,), jnp.int32),
            pltpu.VMEM((EB, H), jnp.float32),
            pltpu.VMEM((EB, H), jnp.float32),
            pltpu.VMEM_SHARED((NG, H), jnp.float32),
            pltpu.VMEM_SHARED((NG, H), jnp.float32),
            pltpu.SemaphoreType.DMA,
            pltpu.SemaphoreType.DMA,
            pltpu.SemaphoreType.DMA,
            pltpu.SemaphoreType.DMA,
            pltpu.SemaphoreType.DMA,
            pltpu.SemaphoreType.DMA,
        ],
    )
    return f(zg, zs, g_src.reshape(NW, NBG, 1, EB), g_dst.reshape(NW, NBG, 1, EB),
             s_src.reshape(NW, NBG, 1, EB), s_dst.reshape(NW, NBG, 1, EB), zero_ng)


def _rgcn_linear_body(agg_ref, deg_ref, ent_ref, wkg_ref, wself_ref, out_ref):
    deg = jnp.maximum(deg_ref[0, :, 0:1] + deg_ref[1, :, 0:1], 1.0)
    agg = (agg_ref[0] + agg_ref[1]) / deg
    h = jnp.dot(agg, wkg_ref[...], preferred_element_type=jnp.float32)
    h = h + jnp.dot(ent_ref[...], wself_ref[...], preferred_element_type=jnp.float32)
    out_ref[...] = jnp.maximum(h, 0.0)


def _rgcn_linear(agg_part, dkg_part, ent_emb, W_kg, W_self):
    grid = 10
    bm = N_ENTS // grid
    return pl.pallas_call(
        _rgcn_linear_body,
        grid=(grid,),
        in_specs=[
            pl.BlockSpec((NC, bm, H), lambda i: (0, i, 0)),
            pl.BlockSpec((NC, bm, H), lambda i: (0, i, 0)),
            pl.BlockSpec((bm, H), lambda i: (i, 0)),
            pl.BlockSpec((H, H), lambda i: (0, 0)),
            pl.BlockSpec((H, H), lambda i: (0, 0)),
        ],
        out_specs=pl.BlockSpec((bm, H), lambda i: (i, 0)),
        out_shape=jax.ShapeDtypeStruct((N_ENTS, H), jnp.float32),
    )(agg_part, dkg_part, ent_emb, W_kg, W_self)


def _rsq_body(dg_ref, ds_ref, og_ref, os_ref):
    og_ref[...] = jax.lax.rsqrt(jnp.maximum(dg_ref[0, :, 0:1] + dg_ref[1, :, 0:1], 1.0))
    os_ref[...] = jax.lax.rsqrt(jnp.maximum(ds_ref[0, :, 0:1] + ds_ref[1, :, 0:1], 1.0))


def _rsq_degs(dgg_part, dgs_part):
    return pl.pallas_call(
        _rsq_body,
        out_shape=(
            jax.ShapeDtypeStruct((NG, 1), jnp.float32),
            jax.ShapeDtypeStruct((NG, 1), jnp.float32),
        ),
    )(dgg_part, dgs_part)


def _pool_scale_body(y2x_ref, h_ref, rg_ref, rs_ref, out_ref):
    blk = y2x_ref[...]
    y = jnp.dot(blk, h_ref[...], preferred_element_type=jnp.float32)
    rs = jnp.clip(jnp.sum(blk, axis=1, keepdims=True), 1e-8, None)
    g = y / rs
    out_ref[0] = g * rg_ref[...]
    out_ref[1] = g * rs_ref[...]


def _pool_g2o_scaled(g2o, kg_h, rsq_g, rsq_s):
    grid = 10
    bm = NG // grid
    return pl.pallas_call(
        _pool_scale_body,
        grid=(grid,),
        in_specs=[
            pl.BlockSpec((bm, N_ENTS), lambda i: (i, 0)),
            pl.BlockSpec((N_ENTS, H), lambda i: (0, 0)),
            pl.BlockSpec((bm, 1), lambda i: (i, 0)),
            pl.BlockSpec((bm, 1), lambda i: (i, 0)),
        ],
        out_specs=pl.BlockSpec((2, bm, H), lambda i: (0, i, 0)),
        out_shape=jax.ShapeDtypeStruct((2, NG, H), jnp.float32),
    )(g2o, kg_h, rsq_g, rsq_s)


def _gcn_mid_body(u1_ref, rg_ref, rs_ref, w1_ref, out_ref):
    w1 = w1_ref[...]
    rg = rg_ref[...]
    rs = rs_ref[...]
    ug = u1_ref[0, 0] + u1_ref[1, 0]
    us = u1_ref[0, 1] + u1_ref[1, 1]
    hg = jnp.maximum(jnp.dot(ug * rg, w1, preferred_element_type=jnp.float32), 0.0)
    hs = jnp.maximum(jnp.dot(us * rs, w1, preferred_element_type=jnp.float32), 0.0)
    out_ref[0] = hg * rg
    out_ref[1] = hs * rs


def _gcn_mid(u1_part, rsq_g, rsq_s, W1):
    return pl.pallas_call(
        _gcn_mid_body,
        out_shape=jax.ShapeDtypeStruct((2, NG, H), jnp.float32),
    )(u1_part, rsq_g, rsq_s, W1)


def _leaky(x):
    return jnp.where(x >= 0, x, 0.01 * x)


def _l2n(x):
    n = jnp.sqrt(jnp.sum(x * x, axis=1, keepdims=True))
    return x / jnp.maximum(n, 1e-12)


def _nce_loss(x1, x2):
    x1 = _l2n(x1)
    x2 = _l2n(x2)
    sm = jnp.exp(jnp.dot(x1, x2.T, preferred_element_type=jnp.float32) / TAU)
    s = jnp.sum(sm, axis=1)
    n = x1.shape[0]
    eye = (jax.lax.broadcasted_iota(jnp.int32, (n, n), 0)
           == jax.lax.broadcasted_iota(jnp.int32, (n, n), 1))
    pos = jnp.sum(jnp.where(eye, sm, 0.0), axis=1)
    return -jnp.mean(jnp.log(pos / s + 1e-8))


def _final_body(u2_ref, rg_ref, rs_ref, w2_ref, d2g_ref, dh2_ref, wp1_ref,
                bp1_ref, wp2_ref, bp2_ref, idx_ref, st_ref, lam_ref,
                dz_ref, demb_ref, sim_ref, loss_ref):
    w2 = w2_ref[...]
    ug = u2_ref[0, 0] + u2_ref[1, 0]
    us = u2_ref[0, 1] + u2_ref[1, 1]
    g_h1 = jnp.dot(ug * rg_ref[...], w2, preferred_element_type=jnp.float32)
    g_h2 = jnp.dot(us * rs_ref[...], w2, preferred_element_type=jnp.float32)
    g_havg = 0.5 * (g_h1 + g_h2)

    d2g = d2g_ref[...]
    rs = jnp.clip(jnp.sum(d2g, axis=1, keepdims=True), 1e-8, None)
    d_h1 = jnp.dot(d2g, g_havg, preferred_element_type=jnp.float32) / rs
    d_emb = jnp.dot(d_h1, wp1_ref[...], preferred_element_type=jnp.float32) + bp1_ref[...]
    d_z2p = jnp.dot(dh2_ref[...], wp2_ref[...], preferred_element_type=jnp.float32) + bp2_ref[...]

    idx = idx_ref[...]
    col = jax.lax.broadcasted_iota(jnp.int32, (B, ND), 1)
    oh1 = (idx[:, 0:1] == col).astype(jnp.float32)
    oh2 = (idx[:, 1:2] == col).astype(jnp.float32)
    d_z1 = _leaky(jnp.dot(oh1, d_emb, preferred_element_type=jnp.float32))
    d_z2 = _leaky(jnp.dot(oh2, d_z2p, preferred_element_type=jnp.float32))
    d_z = 0.5 * (d_z1 + d_z2)

    sim = _l2n(jnp.dot(d_z, d_z.T, preferred_element_type=jnp.float32))
    logits = sim + 1e-8
    mx = jnp.max(logits, axis=1, keepdims=True)
    lse = mx + jnp.log(jnp.sum(jnp.exp(logits - mx), axis=1, keepdims=True))
    logp = logits - lse
    colb = jax.lax.broadcasted_iota(jnp.int32, (B, B), 1)
    oht = (st_ref[...][:, 0:1] == colb)
    l_p = -jnp.mean(jnp.sum(jnp.where(oht, logp, 0.0), axis=1))

    l_g = _nce_loss(g_h1, g_h2)
    l_d = _nce_loss(d_z1, d_z2)
    lam = lam_ref[...]
    loss = l_p + lam[0, 0] * l_g + lam[0, 1] * l_d

    dz_ref[...] = d_z
    demb_ref[...] = d_emb
    sim_ref[...] = sim
    loss_ref[...] = jnp.reshape(loss, (1, 1))


def _final_block(u2_part, rsq_g, rsq_s, W2, d2g, d_h2, Wp1, bp1, Wp2, bp2,
                 index, similarity_true, lam):
    return pl.pallas_call(
        _final_body,
        out_shape=(
            jax.ShapeDtypeStruct((B, ZD), jnp.float32),
            jax.ShapeDtypeStruct((ND, ZD), jnp.float32),
            jax.ShapeDtypeStruct((B, B), jnp.float32),
            jax.ShapeDtypeStruct((1, 1), jnp.float32),
        ),
    )(u2_part, rsq_g, rsq_s, W2, d2g, d_h2,
      Wp1, bp1.reshape(1, ZD), Wp2, bp2.reshape(1, ZD),
      index, similarity_true.reshape(B, 1), lam)


def kernel(g_edge_index, g_svd_edge_index, kg_edge_index, kg_edge_type, g2o,
           d2g, d_h2, index, similarity_true, lam_1, lam_2, ent_emb, rel_emb,
           W_kg, W_self, W1, W2, Wp1, bp1, Wp2, bp2):
    g_src, g_dst = g_edge_index[0], g_edge_index[1]
    s_src, s_dst = g_svd_edge_index[0], g_svd_edge_index[1]
    agg_part, dkg_part, dgg_part, dgs_part = _rgcn_sc(
        kg_edge_index[0], kg_edge_index[1], kg_edge_type, g_dst, s_dst,
        ent_emb, rel_emb)

    kg_h = _rgcn_linear(agg_part, dkg_part, ent_emb, W_kg, W_self)
    rsq_g, rsq_s = _rsq_degs(dgg_part, dgs_part)

    z0 = _pool_g2o_scaled(g2o, kg_h, rsq_g, rsq_s)
    u1_part = _gcn_agg(z0[0], z0[1], g_src, g_dst, s_src, s_dst)
    z1 = _gcn_mid(u1_part, rsq_g, rsq_s, W1)
    u2_part = _gcn_agg(z1[0], z1[1], g_src, g_dst, s_src, s_dst)

    lam = jnp.stack([jnp.squeeze(lam_1), jnp.squeeze(lam_2)]).reshape(1, 2)
    d_z, d_emb, sim, loss = _final_block(
        u2_part, rsq_g, rsq_s, W2, d2g, d_h2, Wp1, bp1, Wp2, bp2,
        index, similarity_true, lam)
    return (d_z, d_emb, sim, jnp.reshape(loss, ()))
